# baseline (speedup 1.0000x reference)
import jax
import jax.numpy as jnp
from jax import lax
from jax.experimental import pallas as pl
from jax.experimental.pallas import tpu as pltpu
from jax.experimental.pallas import tpu_sc as plsc

D_MODEL = 1024
BATCH = 4
TOTAL = 4096

NUM_CORES = 2
NUM_SUBCORES = 16
NUM_WORKERS = NUM_CORES * NUM_SUBCORES
POS_PER_W = TOTAL // NUM_WORKERS

C = 8
NBUF = 3
NCHUNK = POS_PER_W // C
LANES = 16


def _addupdate(ref, x):
  plsc.addupdate(ref, x)


def _body(x_hbm, emb_hbm, ids_hbm, out_hbm,
          ids_v, emb_v, buf0, buf1, buf2,
          si0, si1, si2, so0, so1, so2):
  wid = lax.axis_index("s") * NUM_CORES + lax.axis_index("c")
  base = wid * POS_PER_W

  pltpu.sync_copy(ids_hbm.at[pl.ds(base, POS_PER_W)], ids_v)
  pltpu.sync_copy(emb_hbm, emb_v)

  bufs = (buf0, buf1, buf2)
  sems_in = (si0, si1, si2)
  sems_out = (so0, so1, so2)

  def in_copy(g, b):
    pos0 = base + jnp.maximum(g, 0) * C
    return pltpu.make_async_copy(x_hbm.at[pl.ds(pos0, C)],
                                 bufs[b], sems_in[b])

  def out_copy(g, b):
    pos0 = base + jnp.maximum(g, 0) * C
    return pltpu.make_async_copy(bufs[b],
                                 out_hbm.at[pl.ds(pos0, C)], sems_out[b])

  in_copy(0, 0).start()
  in_copy(1, 1).start()

  iota = lax.iota(jnp.int32, LANES)

  def compute_chunk(g, buf):
    idv = [
        plsc.load_gather(ids_v, [jnp.full((LANES,), g * C + p, jnp.int32)])
        for p in range(C)
    ]

    @plsc.parallel_loop(0, D_MODEL // LANES, unroll=4)
    def _(j):
      col = j * LANES + iota
      for p in range(C):
        ev = plsc.load_gather(emb_v, [idv[p], col])
        for bb in range(BATCH):
          _addupdate(buf.at[p, bb, pl.ds(j * LANES, LANES)], ev)

  def step(k, _):
    for b in range(NBUF):
      g = k * NBUF + b
      in_copy(g, b).wait()
      compute_chunk(g, bufs[b])
      out_copy(g, b).start()

      @pl.when(g + 2 < NCHUNK)
      def _():
        @pl.when(g >= 1)
        def _():
          out_copy(g - 1, (b + 2) % NBUF).wait()

        in_copy(g + 2, (b + 2) % NBUF).start()
    return 0

  lax.fori_loop(0, (NCHUNK - 1) // NBUF, step, 0)

  g = NCHUNK - 1
  in_copy(g, g % NBUF).wait()
  compute_chunk(g, bufs[g % NBUF])
  out_copy(g, g % NBUF).start()

  for gg in range(NCHUNK - 3, NCHUNK):
    out_copy(gg, gg % NBUF).wait()


@jax.jit
def _run(x, emb, src_ids):
  mesh = plsc.VectorSubcoreMesh(core_axis_name="c", subcore_axis_name="s")
  return pl.kernel(
      _body,
      out_type=jax.ShapeDtypeStruct((TOTAL, BATCH, D_MODEL), jnp.float32),
      mesh=mesh,
      compiler_params=pltpu.CompilerParams(
          needs_layout_passes=False, use_tc_tiling_on_sc=True),
      scratch_types=[
          pltpu.VMEM((POS_PER_W,), jnp.int32),
          pltpu.VMEM((BATCH, D_MODEL), jnp.float32),
          pltpu.VMEM((C, BATCH, D_MODEL), jnp.float32),
          pltpu.VMEM((C, BATCH, D_MODEL), jnp.float32),
          pltpu.VMEM((C, BATCH, D_MODEL), jnp.float32),
          pltpu.SemaphoreType.DMA,
          pltpu.SemaphoreType.DMA,
          pltpu.SemaphoreType.DMA,
          pltpu.SemaphoreType.DMA,
          pltpu.SemaphoreType.DMA,
          pltpu.SemaphoreType.DMA,
      ],
  )(x, emb, src_ids)


def kernel(x, emb, src_ids):
  return _run(x, emb, src_ids)


# speedup vs baseline: 1.0378x; 1.0378x over previous
import jax
import jax.numpy as jnp
from jax import lax
from jax.experimental import pallas as pl
from jax.experimental.pallas import tpu as pltpu
from jax.experimental.pallas import tpu_sc as plsc

D_MODEL = 1024
BATCH = 4
TOTAL = 4096

NUM_CORES = 2
NUM_SUBCORES = 16
NUM_WORKERS = NUM_CORES * NUM_SUBCORES
POS_PER_W = TOTAL // NUM_WORKERS

C = 8
NBUF = 3
NCHUNK = POS_PER_W // C
LANES = 16


def _addupdate(ref, x):
  plsc.addupdate(ref, x)


def _body(x_hbm, emb_hbm, ids_hbm, out_hbm,
          ids_v, emb_v, buf0, buf1, buf2,
          si0, si1, si2, so0, so1, so2):
  wid = lax.axis_index("s") * NUM_CORES + lax.axis_index("c")
  base = wid * POS_PER_W

  bufs = (buf0, buf1, buf2)
  sems_in = (si0, si1, si2)
  sems_out = (so0, so1, so2)

  def in_copy(g, b):
    pos0 = base + jnp.maximum(g, 0) * C
    return pltpu.make_async_copy(x_hbm.at[pl.ds(pos0, C)],
                                 bufs[b], sems_in[b])

  def out_copy(g, b):
    pos0 = base + jnp.maximum(g, 0) * C
    return pltpu.make_async_copy(bufs[b],
                                 out_hbm.at[pl.ds(pos0, C)], sems_out[b])

  in_copy(0, 0).start()
  in_copy(1, 1).start()
  stage_ids = pltpu.make_async_copy(
      ids_hbm.at[pl.ds(base, POS_PER_W)], ids_v, so0)
  stage_emb = pltpu.make_async_copy(emb_hbm, emb_v, so1)
  stage_ids.start()
  stage_emb.start()
  stage_ids.wait()
  stage_emb.wait()

  iota = lax.iota(jnp.int32, LANES)

  def compute_chunk(g, buf):
    idv = [
        plsc.load_gather(ids_v, [jnp.full((LANES,), g * C + p, jnp.int32)])
        for p in range(C)
    ]

    @plsc.parallel_loop(0, D_MODEL // LANES, unroll=2)
    def _(j):
      col = j * LANES + iota
      for p in range(C):
        ev = plsc.load_gather(emb_v, [idv[p], col])
        for bb in range(BATCH):
          _addupdate(buf.at[p, bb, pl.ds(j * LANES, LANES)], ev)

  def step(k, _):
    for b in range(NBUF):
      g = k * NBUF + b
      in_copy(g, b).wait()
      compute_chunk(g, bufs[b])
      out_copy(g, b).start()

      @pl.when(g + 2 < NCHUNK)
      def _():
        @pl.when(g >= 1)
        def _():
          out_copy(g - 1, (b + 2) % NBUF).wait()

        in_copy(g + 2, (b + 2) % NBUF).start()
    return 0

  lax.fori_loop(0, (NCHUNK - 1) // NBUF, step, 0)

  g = NCHUNK - 1
  in_copy(g, g % NBUF).wait()
  compute_chunk(g, bufs[g % NBUF])
  out_copy(g, g % NBUF).start()

  for gg in range(NCHUNK - 3, NCHUNK):
    out_copy(gg, gg % NBUF).wait()


@jax.jit
def _run(x, emb, src_ids):
  mesh = plsc.VectorSubcoreMesh(core_axis_name="c", subcore_axis_name="s")
  return pl.kernel(
      _body,
      out_type=jax.ShapeDtypeStruct((TOTAL, BATCH, D_MODEL), jnp.float32),
      mesh=mesh,
      compiler_params=pltpu.CompilerParams(
          needs_layout_passes=False, use_tc_tiling_on_sc=True),
      scratch_types=[
          pltpu.VMEM((POS_PER_W,), jnp.int32),
          pltpu.VMEM((BATCH, D_MODEL), jnp.float32),
          pltpu.VMEM((C, BATCH, D_MODEL), jnp.float32),
          pltpu.VMEM((C, BATCH, D_MODEL), jnp.float32),
          pltpu.VMEM((C, BATCH, D_MODEL), jnp.float32),
          pltpu.SemaphoreType.DMA,
          pltpu.SemaphoreType.DMA,
          pltpu.SemaphoreType.DMA,
          pltpu.SemaphoreType.DMA,
          pltpu.SemaphoreType.DMA,
          pltpu.SemaphoreType.DMA,
      ],
  )(x, emb, src_ids)


def kernel(x, emb, src_ids):
  return _run(x, emb, src_ids)


# speedup vs baseline: 1.0441x; 1.0061x over previous
import jax
import jax.numpy as jnp
from jax import lax
from jax.experimental import pallas as pl
from jax.experimental.pallas import tpu as pltpu
from jax.experimental.pallas import tpu_sc as plsc

D_MODEL = 1024
BATCH = 4
TOTAL = 4096

NUM_CORES = 2
NUM_SUBCORES = 16
NUM_WORKERS = NUM_CORES * NUM_SUBCORES
POS_PER_W = TOTAL // NUM_WORKERS

C = 8
NBUF = 3
NCHUNK = POS_PER_W // C
LANES = 16


def _addupdate(ref, x):
  plsc.addupdate(ref, x)


def _body(x_hbm, emb_hbm, ids_hbm, out_hbm,
          ids_v, emb_v, buf0, buf1, buf2,
          si0, si1, si2, so0, so1, so2):
  wid = lax.axis_index("s") * NUM_CORES + lax.axis_index("c")
  base = wid * POS_PER_W

  bufs = (buf0, buf1, buf2)
  sems_in = (si0, si1, si2)
  sems_out = (so0, so1, so2)

  def in_copy(g, b):
    pos0 = base + jnp.maximum(g, 0) * C
    return pltpu.make_async_copy(x_hbm.at[pl.ds(pos0, C)],
                                 bufs[b], sems_in[b])

  def out_copy(g, b):
    pos0 = base + jnp.maximum(g, 0) * C
    return pltpu.make_async_copy(bufs[b],
                                 out_hbm.at[pl.ds(pos0, C)], sems_out[b])

  in_copy(0, 0).start()
  in_copy(1, 1).start()
  stage_ids = pltpu.make_async_copy(
      ids_hbm.at[pl.ds(base, POS_PER_W)], ids_v, so0)
  stage_emb = pltpu.make_async_copy(emb_hbm, emb_v, so1)
  stage_ids.start()
  stage_emb.start()
  stage_ids.wait()
  stage_emb.wait()

  iota = lax.iota(jnp.int32, LANES)

  def compute_chunk(g, buf):
    idv = [
        plsc.load_gather(ids_v, [jnp.full((LANES,), g * C + p, jnp.int32)])
        for p in range(C)
    ]

    @plsc.parallel_loop(0, D_MODEL // LANES, unroll=1)
    def _(j):
      col = j * LANES + iota
      for p in range(C):
        ev = plsc.load_gather(emb_v, [idv[p], col])
        for bb in range(BATCH):
          _addupdate(buf.at[p, bb, pl.ds(j * LANES, LANES)], ev)

  def step(k, _):
    for b in range(NBUF):
      g = k * NBUF + b
      in_copy(g, b).wait()
      compute_chunk(g, bufs[b])
      out_copy(g, b).start()

      @pl.when(g + 2 < NCHUNK)
      def _():
        @pl.when(g >= 1)
        def _():
          out_copy(g - 1, (b + 2) % NBUF).wait()

        in_copy(g + 2, (b + 2) % NBUF).start()
    return 0

  lax.fori_loop(0, (NCHUNK - 1) // NBUF, step, 0)

  g = NCHUNK - 1
  in_copy(g, g % NBUF).wait()
  compute_chunk(g, bufs[g % NBUF])
  out_copy(g, g % NBUF).start()

  for gg in range(NCHUNK - 3, NCHUNK):
    out_copy(gg, gg % NBUF).wait()


@jax.jit
def _run(x, emb, src_ids):
  mesh = plsc.VectorSubcoreMesh(core_axis_name="c", subcore_axis_name="s")
  return pl.kernel(
      _body,
      out_type=jax.ShapeDtypeStruct((TOTAL, BATCH, D_MODEL), jnp.float32),
      mesh=mesh,
      compiler_params=pltpu.CompilerParams(
          needs_layout_passes=False, use_tc_tiling_on_sc=True),
      scratch_types=[
          pltpu.VMEM((POS_PER_W,), jnp.int32),
          pltpu.VMEM((BATCH, D_MODEL), jnp.float32),
          pltpu.VMEM((C, BATCH, D_MODEL), jnp.float32),
          pltpu.VMEM((C, BATCH, D_MODEL), jnp.float32),
          pltpu.VMEM((C, BATCH, D_MODEL), jnp.float32),
          pltpu.SemaphoreType.DMA,
          pltpu.SemaphoreType.DMA,
          pltpu.SemaphoreType.DMA,
          pltpu.SemaphoreType.DMA,
          pltpu.SemaphoreType.DMA,
          pltpu.SemaphoreType.DMA,
      ],
  )(x, emb, src_ids)


def kernel(x, emb, src_ids):
  return _run(x, emb, src_ids)
